# baseline (speedup 1.0000x reference)
import functools

import jax
import jax.numpy as jnp
from jax import lax
from jax.experimental import pallas as pl
from jax.experimental.pallas import tpu as pltpu
from jax.experimental.pallas import tpu_sc as plsc

_B = 4
_C = 8
_E = 8
_OUT = 8
_H = 512
_W = 512
_NEG = -3.0e38

_H_SC = 512
_H_TC = _H - _H_SC
_SC_CHUNK = 4
_NW = 32


def _tc_body(gw_ref, gb_ref, ew_ref, eb_ref, x_ref, final_ref, logits_ref):
    xs = [x_ref[0, c] for c in range(_C)]

    xb = [v.astype(jnp.bfloat16).astype(jnp.float32) for v in xs]
    ls = []
    for c in range(_C):
        acc = jnp.full_like(xs[0], gb_ref[0, c])
        for k in range(_C):
            gwk = gw_ref[c, k].astype(jnp.bfloat16).astype(jnp.float32)
            acc = acc + gwk * xb[k]
        ls.append(acc)
        logits_ref[0, c] = acc

    m1 = ls[0]
    for c in range(1, _C):
        m1 = jnp.maximum(m1, ls[c])
    t1 = []
    found = None
    for c in range(_C):
        eq = ls[c] == m1
        if found is None:
            t1.append(eq)
            found = eq
        else:
            t1.append(eq & (~found))
            found = found | eq
    masked = [jnp.where(t1[c], _NEG, ls[c]) for c in range(_C)]
    m2 = masked[0]
    for c in range(1, _C):
        m2 = jnp.maximum(m2, masked[c])
    t2 = []
    found = None
    for c in range(_C):
        eq = masked[c] == m2
        if found is None:
            t2.append(eq)
            found = eq
        else:
            t2.append(eq & (~found))
            found = found | eq

    w2 = 1.0 / (1.0 + jnp.exp(m1 - m2))
    w1 = 1.0 - w2
    zero = jnp.zeros_like(w1)
    ce = [jnp.where(t1[c], w1, jnp.where(t2[c], w2, zero)) for c in range(_C)]

    xp = [v.astype(jnp.bfloat16) for v in xs]
    fin = [None] * _OUT
    for e in range(_E):
        for o in range(_OUT):
            y = ew_ref[e * _OUT + o, 0].astype(jnp.bfloat16) * xp[0]
            for k in range(1, _C):
                y = y + ew_ref[e * _OUT + o, k].astype(jnp.bfloat16) * xp[k]
            y = y + eb_ref[e, o].astype(jnp.bfloat16)
            contrib = ce[e] * y.astype(jnp.float32)
            fin[o] = contrib if fin[o] is None else fin[o] + contrib
    for o in range(_OUT):
        final_ref[0, o] = fin[o]


def _run_tc(x, gw, gb, ew, eb, h_rows, hb=32):
    B, C, H, W = x.shape
    grid = (B, h_rows // hb)
    smem = functools.partial(pl.BlockSpec, memory_space=pltpu.SMEM)
    out_shape = [
        jax.ShapeDtypeStruct((B, _OUT, h_rows, W), x.dtype),
        jax.ShapeDtypeStruct((B, C, h_rows, W), jnp.float32),
    ]
    f = pl.pallas_call(
        _tc_body,
        grid=grid,
        in_specs=[
            smem((C, C), lambda b, h: (0, 0)),
            smem((1, C), lambda b, h: (0, 0)),
            smem((_E * _OUT, C), lambda b, h: (0, 0)),
            smem((_E, _OUT), lambda b, h: (0, 0)),
            pl.BlockSpec((1, C, hb, W), lambda b, h: (b, 0, h, 0)),
        ],
        out_specs=[
            pl.BlockSpec((1, _OUT, hb, W), lambda b, h: (b, 0, h, 0)),
            pl.BlockSpec((1, C, hb, W), lambda b, h: (b, 0, h, 0)),
        ],
        out_shape=out_shape,
    )
    return f(gw, gb, ew, eb, x[:, :, :h_rows])


_EW_BASE = 1152
_EB_BASE = 1664
_WT_PAD = 1728


def _bf16_round(v):
    u = lax.bitcast_convert_type(v, jnp.uint32)
    r = u + jnp.uint32(0x7FFF) + ((u >> 16) & jnp.uint32(1))
    return lax.bitcast_convert_type(r & jnp.uint32(0xFFFF0000), jnp.float32)


def _sc_compute16(wt, gws, gbs, xs):
    xr = [_bf16_round(v) for v in xs]
    ls = []
    for c in range(_C):
        acc = gbs[c]
        for k in range(_C):
            acc = acc + gws[c * _C + k] * xr[k]
        ls.append(acc)

    m1 = ls[0]
    for c in range(1, _C):
        m1 = jnp.maximum(m1, ls[c])
    t1 = []
    found = None
    for c in range(_C):
        eq = ls[c] == m1
        if found is None:
            t1.append(eq)
            found = eq
        else:
            t1.append(eq & (~found))
            found = found | eq
    masked = [jnp.where(t1[c], _NEG, ls[c]) for c in range(_C)]
    m2 = masked[0]
    for c in range(1, _C):
        m2 = jnp.maximum(m2, masked[c])
    t2 = []
    found = None
    for c in range(_C):
        eq = masked[c] == m2
        if found is None:
            t2.append(eq)
            found = eq
        else:
            t2.append(eq & (~found))
            found = found | eq

    w2 = 1.0 / (1.0 + jnp.exp(m1 - m2))
    w1 = 1.0 - w2

    zi = jnp.zeros((16,), jnp.int32)
    e1x64 = zi
    e2x64 = zi
    for c in range(_C):
        e1x64 = e1x64 + jnp.where(t1[c], jnp.int32(c * 64), zi)
        e2x64 = e2x64 + jnp.where(t2[c], jnp.int32(c * 64), zi)
    e1x8 = lax.shift_right_logical(e1x64, 3)
    e2x8 = lax.shift_right_logical(e2x64, 3)

    fin = []
    for o in range(_OUT):
        a1 = plsc.load_gather(wt, [e1x8 + jnp.int32(_EB_BASE + o)])
        a2 = plsc.load_gather(wt, [e2x8 + jnp.int32(_EB_BASE + o)])
        for k in range(_C):
            g1 = plsc.load_gather(wt, [e1x64 + jnp.int32(_EW_BASE + o * 8 + k)])
            g2 = plsc.load_gather(wt, [e2x64 + jnp.int32(_EW_BASE + o * 8 + k)])
            a1 = a1 + g1 * xs[k]
            a2 = a2 + g2 * xs[k]
        fin.append(w1 * a1 + w2 * a2)
    return ls, fin


def _make_sc(h_sc, h0):
    rows_total = _B * h_sc
    rpw = rows_total // _NW
    ch = min(_SC_CHUNK, rpw)
    nchunks = rpw // ch
    npix = ch * _W
    mesh = plsc.VectorSubcoreMesh(core_axis_name="c", subcore_axis_name="s")

    @functools.partial(
        pl.kernel,
        out_type=[
            jax.ShapeDtypeStruct((_B, _OUT, h_sc * _W), jnp.float32),
            jax.ShapeDtypeStruct((_B, _C, h_sc * _W), jnp.float32),
        ],
        mesh=mesh,
        compiler_params=pltpu.CompilerParams(needs_layout_passes=False),
        scratch_types=[
            pltpu.VMEM((_WT_PAD,), jnp.float32),
            pltpu.VMEM((_C, npix), jnp.float32),
            pltpu.VMEM((_OUT, npix), jnp.float32),
            pltpu.VMEM((_C, npix), jnp.float32),
        ],
    )
    def sc_kernel(x_hbm, wt_hbm, fin_hbm, log_hbm, wt, xb, fb, lb):
        wid = lax.axis_index("s") * 2 + lax.axis_index("c")
        pltpu.sync_copy(wt_hbm, wt)
        gws = [wt[pl.ds(i * 16, 16)] for i in range(64)]
        gbs = [wt[pl.ds(1024 + c * 16, 16)] for c in range(_C)]

        def chunk_body(chunk, carry):
            grow = wid * rpw + chunk * ch
            b = grow // h_sc
            r = grow % h_sc
            for k in range(_C):
                pltpu.sync_copy(
                    x_hbm.at[b, k, pl.ds((h0 + r) * _W, npix)], xb.at[k])

            def body(i, c2):
                off = i * 16
                xs = [xb[k, pl.ds(off, 16)] for k in range(_C)]
                ls, fin = _sc_compute16(wt, gws, gbs, xs)
                for c in range(_C):
                    lb[c, pl.ds(off, 16)] = ls[c]
                for o in range(_OUT):
                    fb[o, pl.ds(off, 16)] = fin[o]
                return c2

            lax.fori_loop(0, npix // 16, body, 0)
            for o in range(_OUT):
                pltpu.sync_copy(fb.at[o], fin_hbm.at[b, o, pl.ds(r * _W, npix)])
            for c in range(_C):
                pltpu.sync_copy(lb.at[c], log_hbm.at[b, c, pl.ds(r * _W, npix)])
            return carry

        lax.fori_loop(0, nchunks, chunk_body, 0)

    return sc_kernel


def _pack_weights(gw, gb, ew, eb):
    gw_s = jnp.repeat(_bf16_round(gw.reshape(-1)), 16)
    gb_s = jnp.repeat(gb.reshape(-1), 16)
    return jnp.concatenate([gw_s, gb_s, ew.reshape(-1), eb.reshape(-1)])


@jax.jit
def _run(x, gw, gb, ew, eb):
    outs = []
    if _H_TC > 0:
        outs.append(_run_tc(x, gw, gb, ew, eb, _H_TC))
    if _H_SC > 0:
        wt = _pack_weights(gw, gb, ew, eb)
        sc = _make_sc(_H_SC, _H_TC)
        B, C, H, W = x.shape
        fin_sc, log_sc = sc(x.reshape(B, C, H * W), wt)
        fin_sc = fin_sc.reshape(B, _OUT, _H_SC, W)
        log_sc = log_sc.reshape(B, C, _H_SC, W)
        outs.append((fin_sc, log_sc))
    if len(outs) == 1:
        return outs[0]
    fin = jnp.concatenate([outs[0][0], outs[1][0]], axis=2)
    log = jnp.concatenate([outs[0][1], outs[1][1]], axis=2)
    return fin, log


def kernel(x, gate_w, gate_b, expert_w, expert_b):
    gw = gate_w.reshape(_C, _C)
    gb = gate_b.reshape(1, _C)
    ew = expert_w.reshape(_E * _OUT, _C)
    eb = expert_b.reshape(_E, _OUT)
    final, logits = _run(x, gw, gb, ew, eb)
    return (final, logits)


# speedup vs baseline: 1.0349x; 1.0349x over previous
import functools

import jax
import jax.numpy as jnp
from jax import lax
from jax.experimental import pallas as pl
from jax.experimental.pallas import tpu as pltpu
from jax.experimental.pallas import tpu_sc as plsc

_B = 4
_C = 8
_E = 8
_OUT = 8
_H = 512
_W = 512
_NEG = -3.0e38

_H_SC = 512
_H_TC = _H - _H_SC
_SC_CHUNK = 8
_NW = 32


def _tc_body(gw_ref, gb_ref, ew_ref, eb_ref, x_ref, final_ref, logits_ref):
    xs = [x_ref[0, c] for c in range(_C)]

    xb = [v.astype(jnp.bfloat16).astype(jnp.float32) for v in xs]
    ls = []
    for c in range(_C):
        acc = jnp.full_like(xs[0], gb_ref[0, c])
        for k in range(_C):
            gwk = gw_ref[c, k].astype(jnp.bfloat16).astype(jnp.float32)
            acc = acc + gwk * xb[k]
        ls.append(acc)
        logits_ref[0, c] = acc

    m1 = ls[0]
    for c in range(1, _C):
        m1 = jnp.maximum(m1, ls[c])
    t1 = []
    found = None
    for c in range(_C):
        eq = ls[c] == m1
        if found is None:
            t1.append(eq)
            found = eq
        else:
            t1.append(eq & (~found))
            found = found | eq
    masked = [jnp.where(t1[c], _NEG, ls[c]) for c in range(_C)]
    m2 = masked[0]
    for c in range(1, _C):
        m2 = jnp.maximum(m2, masked[c])
    t2 = []
    found = None
    for c in range(_C):
        eq = masked[c] == m2
        if found is None:
            t2.append(eq)
            found = eq
        else:
            t2.append(eq & (~found))
            found = found | eq

    w2 = 1.0 / (1.0 + jnp.exp(m1 - m2))
    w1 = 1.0 - w2
    zero = jnp.zeros_like(w1)
    ce = [jnp.where(t1[c], w1, jnp.where(t2[c], w2, zero)) for c in range(_C)]

    xp = [v.astype(jnp.bfloat16) for v in xs]
    fin = [None] * _OUT
    for e in range(_E):
        for o in range(_OUT):
            y = ew_ref[e * _OUT + o, 0].astype(jnp.bfloat16) * xp[0]
            for k in range(1, _C):
                y = y + ew_ref[e * _OUT + o, k].astype(jnp.bfloat16) * xp[k]
            y = y + eb_ref[e, o].astype(jnp.bfloat16)
            contrib = ce[e] * y.astype(jnp.float32)
            fin[o] = contrib if fin[o] is None else fin[o] + contrib
    for o in range(_OUT):
        final_ref[0, o] = fin[o]


def _run_tc(x, gw, gb, ew, eb, h_rows, hb=32):
    B, C, H, W = x.shape
    grid = (B, h_rows // hb)
    smem = functools.partial(pl.BlockSpec, memory_space=pltpu.SMEM)
    out_shape = [
        jax.ShapeDtypeStruct((B, _OUT, h_rows, W), x.dtype),
        jax.ShapeDtypeStruct((B, C, h_rows, W), jnp.float32),
    ]
    f = pl.pallas_call(
        _tc_body,
        grid=grid,
        in_specs=[
            smem((C, C), lambda b, h: (0, 0)),
            smem((1, C), lambda b, h: (0, 0)),
            smem((_E * _OUT, C), lambda b, h: (0, 0)),
            smem((_E, _OUT), lambda b, h: (0, 0)),
            pl.BlockSpec((1, C, hb, W), lambda b, h: (b, 0, h, 0)),
        ],
        out_specs=[
            pl.BlockSpec((1, _OUT, hb, W), lambda b, h: (b, 0, h, 0)),
            pl.BlockSpec((1, C, hb, W), lambda b, h: (b, 0, h, 0)),
        ],
        out_shape=out_shape,
    )
    return f(gw, gb, ew, eb, x[:, :, :h_rows])


_EW_BASE = 1152
_EB_BASE = 1664
_WT_PAD = 1728


def _bf16_round(v):
    u = lax.bitcast_convert_type(v, jnp.uint32)
    r = u + jnp.uint32(0x7FFF) + ((u >> 16) & jnp.uint32(1))
    return lax.bitcast_convert_type(r & jnp.uint32(0xFFFF0000), jnp.float32)


def _sc_compute16(wt, gws, gbs, xs):
    xr = [_bf16_round(v) for v in xs]
    ls = []
    for c in range(_C):
        acc = gbs[c]
        for k in range(_C):
            acc = acc + gws[c * _C + k] * xr[k]
        ls.append(acc)

    m1 = ls[0]
    for c in range(1, _C):
        m1 = jnp.maximum(m1, ls[c])
    t1 = []
    found = None
    for c in range(_C):
        eq = ls[c] == m1
        if found is None:
            t1.append(eq)
            found = eq
        else:
            t1.append(eq & (~found))
            found = found | eq
    masked = [jnp.where(t1[c], _NEG, ls[c]) for c in range(_C)]
    m2 = masked[0]
    for c in range(1, _C):
        m2 = jnp.maximum(m2, masked[c])
    t2 = []
    found = None
    for c in range(_C):
        eq = masked[c] == m2
        if found is None:
            t2.append(eq)
            found = eq
        else:
            t2.append(eq & (~found))
            found = found | eq

    w2 = 1.0 / (1.0 + jnp.exp(m1 - m2))
    w1 = 1.0 - w2

    zi = jnp.zeros((16,), jnp.int32)
    e1x64 = zi
    e2x64 = zi
    for c in range(_C):
        e1x64 = e1x64 + jnp.where(t1[c], jnp.int32(c * 64), zi)
        e2x64 = e2x64 + jnp.where(t2[c], jnp.int32(c * 64), zi)
    e1x8 = lax.shift_right_logical(e1x64, 3)
    e2x8 = lax.shift_right_logical(e2x64, 3)

    fin = []
    for o in range(_OUT):
        a1 = plsc.load_gather(wt, [e1x8 + jnp.int32(_EB_BASE + o)])
        a2 = plsc.load_gather(wt, [e2x8 + jnp.int32(_EB_BASE + o)])
        for k in range(_C):
            g1 = plsc.load_gather(wt, [e1x64 + jnp.int32(_EW_BASE + o * 8 + k)])
            g2 = plsc.load_gather(wt, [e2x64 + jnp.int32(_EW_BASE + o * 8 + k)])
            a1 = a1 + g1 * xs[k]
            a2 = a2 + g2 * xs[k]
        fin.append(w1 * a1 + w2 * a2)
    return ls, fin


def _make_sc(h_sc, h0):
    rows_total = _B * h_sc
    rpw = rows_total // _NW
    ch = min(_SC_CHUNK, rpw)
    nchunks = rpw // ch
    npix = ch * _W
    mesh = plsc.VectorSubcoreMesh(core_axis_name="c", subcore_axis_name="s")

    @functools.partial(
        pl.kernel,
        out_type=[
            jax.ShapeDtypeStruct((_B, _OUT, h_sc * _W), jnp.float32),
            jax.ShapeDtypeStruct((_B, _C, h_sc * _W), jnp.float32),
        ],
        mesh=mesh,
        compiler_params=pltpu.CompilerParams(needs_layout_passes=False),
        scratch_types=[
            pltpu.VMEM((_WT_PAD,), jnp.float32),
            pltpu.VMEM((_C, npix), jnp.float32),
            pltpu.VMEM((_OUT, npix), jnp.float32),
            pltpu.VMEM((_C, npix), jnp.float32),
        ],
    )
    def sc_kernel(x_hbm, wt_hbm, fin_hbm, log_hbm, wt, xb, fb, lb):
        wid = lax.axis_index("s") * 2 + lax.axis_index("c")
        pltpu.sync_copy(wt_hbm, wt)
        gws = [wt[pl.ds(i * 16, 16)] for i in range(64)]
        gbs = [wt[pl.ds(1024 + c * 16, 16)] for c in range(_C)]

        def chunk_body(chunk, carry):
            grow = wid * rpw + chunk * ch
            b = grow // h_sc
            r = grow % h_sc
            pltpu.sync_copy(x_hbm.at[b, :, pl.ds((h0 + r) * _W, npix)], xb)

            def body(i, c2):
                off = i * 16
                xs = [xb[k, pl.ds(off, 16)] for k in range(_C)]
                ls, fin = _sc_compute16(wt, gws, gbs, xs)
                for c in range(_C):
                    lb[c, pl.ds(off, 16)] = ls[c]
                for o in range(_OUT):
                    fb[o, pl.ds(off, 16)] = fin[o]
                return c2

            lax.fori_loop(0, npix // 16, body, 0)
            pltpu.sync_copy(fb, fin_hbm.at[b, :, pl.ds(r * _W, npix)])
            pltpu.sync_copy(lb, log_hbm.at[b, :, pl.ds(r * _W, npix)])
            return carry

        lax.fori_loop(0, nchunks, chunk_body, 0)

    return sc_kernel


def _pack_weights(gw, gb, ew, eb):
    gw_s = jnp.repeat(_bf16_round(gw.reshape(-1)), 16)
    gb_s = jnp.repeat(gb.reshape(-1), 16)
    return jnp.concatenate([gw_s, gb_s, ew.reshape(-1), eb.reshape(-1)])


@jax.jit
def _run(x, gw, gb, ew, eb):
    outs = []
    if _H_TC > 0:
        outs.append(_run_tc(x, gw, gb, ew, eb, _H_TC))
    if _H_SC > 0:
        wt = _pack_weights(gw, gb, ew, eb)
        sc = _make_sc(_H_SC, _H_TC)
        B, C, H, W = x.shape
        fin_sc, log_sc = sc(x.reshape(B, C, H * W), wt)
        fin_sc = fin_sc.reshape(B, _OUT, _H_SC, W)
        log_sc = log_sc.reshape(B, C, _H_SC, W)
        outs.append((fin_sc, log_sc))
    if len(outs) == 1:
        return outs[0]
    fin = jnp.concatenate([outs[0][0], outs[1][0]], axis=2)
    log = jnp.concatenate([outs[0][1], outs[1][1]], axis=2)
    return fin, log


def kernel(x, gate_w, gate_b, expert_w, expert_b):
    gw = gate_w.reshape(_C, _C)
    gb = gate_b.reshape(1, _C)
    ew = expert_w.reshape(_E * _OUT, _C)
    eb = expert_b.reshape(_E, _OUT)
    final, logits = _run(x, gw, gb, ew, eb)
    return (final, logits)


# speedup vs baseline: 3.7375x; 3.6114x over previous
import functools

import jax
import jax.numpy as jnp
from jax import lax
from jax.experimental import pallas as pl
from jax.experimental.pallas import tpu as pltpu
from jax.experimental.pallas import tpu_sc as plsc

_B = 4
_C = 8
_E = 8
_OUT = 8
_H = 512
_W = 512
_NEG = -3.0e38

_H_SC = 512
_H_TC = _H - _H_SC
_SC_CHUNK = 8
_NW = 32


def _tc_body(gw_ref, gb_ref, ew_ref, eb_ref, x_ref, final_ref, logits_ref):
    xs = [x_ref[0, c] for c in range(_C)]

    xb = [v.astype(jnp.bfloat16).astype(jnp.float32) for v in xs]
    ls = []
    for c in range(_C):
        acc = jnp.full_like(xs[0], gb_ref[0, c])
        for k in range(_C):
            gwk = gw_ref[c, k].astype(jnp.bfloat16).astype(jnp.float32)
            acc = acc + gwk * xb[k]
        ls.append(acc)
        logits_ref[0, c] = acc

    m1 = ls[0]
    for c in range(1, _C):
        m1 = jnp.maximum(m1, ls[c])
    t1 = []
    found = None
    for c in range(_C):
        eq = ls[c] == m1
        if found is None:
            t1.append(eq)
            found = eq
        else:
            t1.append(eq & (~found))
            found = found | eq
    masked = [jnp.where(t1[c], _NEG, ls[c]) for c in range(_C)]
    m2 = masked[0]
    for c in range(1, _C):
        m2 = jnp.maximum(m2, masked[c])
    t2 = []
    found = None
    for c in range(_C):
        eq = masked[c] == m2
        if found is None:
            t2.append(eq)
            found = eq
        else:
            t2.append(eq & (~found))
            found = found | eq

    w2 = 1.0 / (1.0 + jnp.exp(m1 - m2))
    w1 = 1.0 - w2
    zero = jnp.zeros_like(w1)
    ce = [jnp.where(t1[c], w1, jnp.where(t2[c], w2, zero)) for c in range(_C)]

    xp = [v.astype(jnp.bfloat16) for v in xs]
    fin = [None] * _OUT
    for e in range(_E):
        for o in range(_OUT):
            y = ew_ref[e * _OUT + o, 0].astype(jnp.bfloat16) * xp[0]
            for k in range(1, _C):
                y = y + ew_ref[e * _OUT + o, k].astype(jnp.bfloat16) * xp[k]
            y = y + eb_ref[e, o].astype(jnp.bfloat16)
            contrib = ce[e] * y.astype(jnp.float32)
            fin[o] = contrib if fin[o] is None else fin[o] + contrib
    for o in range(_OUT):
        final_ref[0, o] = fin[o]


def _run_tc(x, gw, gb, ew, eb, h_rows, hb=32):
    B, C, H, W = x.shape
    grid = (B, h_rows // hb)
    smem = functools.partial(pl.BlockSpec, memory_space=pltpu.SMEM)
    out_shape = [
        jax.ShapeDtypeStruct((B, _OUT, h_rows, W), x.dtype),
        jax.ShapeDtypeStruct((B, C, h_rows, W), jnp.float32),
    ]
    f = pl.pallas_call(
        _tc_body,
        grid=grid,
        in_specs=[
            smem((C, C), lambda b, h: (0, 0)),
            smem((1, C), lambda b, h: (0, 0)),
            smem((_E * _OUT, C), lambda b, h: (0, 0)),
            smem((_E, _OUT), lambda b, h: (0, 0)),
            pl.BlockSpec((1, C, hb, W), lambda b, h: (b, 0, h, 0)),
        ],
        out_specs=[
            pl.BlockSpec((1, _OUT, hb, W), lambda b, h: (b, 0, h, 0)),
            pl.BlockSpec((1, C, hb, W), lambda b, h: (b, 0, h, 0)),
        ],
        out_shape=out_shape,
    )
    return f(gw, gb, ew, eb, x[:, :, :h_rows])


_EW_BASE = 1152
_EB_BASE = 1672
_WT_PAD = 1744


def _bf16_round(v):
    u = lax.bitcast_convert_type(v, jnp.uint32)
    r = u + jnp.uint32(0x7FFF) + ((u >> 16) & jnp.uint32(1))
    return lax.bitcast_convert_type(r & jnp.uint32(0xFFFF0000), jnp.float32)


def _sc_compute16(wt, gws, gbs, xs):
    xr = [_bf16_round(v) for v in xs]
    ls = []
    for c in range(_C):
        acc = gbs[c]
        for k in range(_C):
            acc = acc + gws[c * _C + k] * xr[k]
        ls.append(acc)

    m1 = ls[0]
    for c in range(1, _C):
        m1 = jnp.maximum(m1, ls[c])
    t1 = []
    found = None
    for c in range(_C):
        eq = ls[c] == m1
        if found is None:
            t1.append(eq)
            found = eq
        else:
            t1.append(eq & (~found))
            found = found | eq
    masked = [jnp.where(t1[c], _NEG, ls[c]) for c in range(_C)]
    m2 = masked[0]
    for c in range(1, _C):
        m2 = jnp.maximum(m2, masked[c])
    t2 = []
    found = None
    for c in range(_C):
        eq = masked[c] == m2
        if found is None:
            t2.append(eq)
            found = eq
        else:
            t2.append(eq & (~found))
            found = found | eq

    w2 = 1.0 / (1.0 + jnp.exp(m1 - m2))
    w1 = 1.0 - w2

    zi = jnp.zeros((16,), jnp.int32)
    e1 = zi
    e2 = zi
    for c in range(1, _C):
        e1 = e1 + jnp.where(t1[c], jnp.int32(c), zi)
        e2 = e2 + jnp.where(t2[c], jnp.int32(c), zi)
    e1x65 = lax.shift_left(e1, 6) + e1
    e2x65 = lax.shift_left(e2, 6) + e2
    e1x9 = lax.shift_left(e1, 3) + e1
    e2x9 = lax.shift_left(e2, 3) + e2

    fin = []
    for o in range(_OUT):
        a1 = plsc.load_gather(wt, [e1x9 + jnp.int32(_EB_BASE + o)])
        a2 = plsc.load_gather(wt, [e2x9 + jnp.int32(_EB_BASE + o)])
        for k in range(_C):
            g1 = plsc.load_gather(wt, [e1x65 + jnp.int32(_EW_BASE + o * 8 + k)])
            g2 = plsc.load_gather(wt, [e2x65 + jnp.int32(_EW_BASE + o * 8 + k)])
            a1 = a1 + g1 * xs[k]
            a2 = a2 + g2 * xs[k]
        fin.append(w1 * a1 + w2 * a2)
    return ls, fin


def _make_sc(h_sc, h0):
    rows_total = _B * h_sc
    rpw = rows_total // _NW
    ch = min(_SC_CHUNK, rpw)
    nchunks = rpw // ch
    npix = ch * _W
    mesh = plsc.VectorSubcoreMesh(core_axis_name="c", subcore_axis_name="s")

    @functools.partial(
        pl.kernel,
        out_type=[
            jax.ShapeDtypeStruct((_B, _OUT, h_sc * _W), jnp.float32),
            jax.ShapeDtypeStruct((_B, _C, h_sc * _W), jnp.float32),
        ],
        mesh=mesh,
        compiler_params=pltpu.CompilerParams(needs_layout_passes=False),
        scratch_types=[
            pltpu.VMEM((_WT_PAD,), jnp.float32),
            pltpu.VMEM((_C, npix), jnp.float32),
            pltpu.VMEM((_OUT, npix), jnp.float32),
            pltpu.VMEM((_C, npix), jnp.float32),
        ],
    )
    def sc_kernel(x_hbm, wt_hbm, fin_hbm, log_hbm, wt, xb, fb, lb):
        wid = lax.axis_index("s") * 2 + lax.axis_index("c")
        pltpu.sync_copy(wt_hbm, wt)
        gws = [wt[pl.ds(i * 16, 16)] for i in range(64)]
        gbs = [wt[pl.ds(1024 + c * 16, 16)] for c in range(_C)]

        def chunk_body(chunk, carry):
            grow = wid * rpw + chunk * ch
            b = grow // h_sc
            r = grow % h_sc
            pltpu.sync_copy(x_hbm.at[b, :, pl.ds((h0 + r) * _W, npix)], xb)

            def body(i, c2):
                off = i * 16
                xs = [xb[k, pl.ds(off, 16)] for k in range(_C)]
                ls, fin = _sc_compute16(wt, gws, gbs, xs)
                for c in range(_C):
                    lb[c, pl.ds(off, 16)] = ls[c]
                for o in range(_OUT):
                    fb[o, pl.ds(off, 16)] = fin[o]
                return c2

            lax.fori_loop(0, npix // 16, body, 0)
            pltpu.sync_copy(fb, fin_hbm.at[b, :, pl.ds(r * _W, npix)])
            pltpu.sync_copy(lb, log_hbm.at[b, :, pl.ds(r * _W, npix)])
            return carry

        lax.fori_loop(0, nchunks, chunk_body, 0)

    return sc_kernel


def _pack_weights(gw, gb, ew, eb):
    gw_s = jnp.repeat(_bf16_round(gw.reshape(-1)), 16)
    gb_s = jnp.repeat(gb.reshape(-1), 16)
    ew_p = jnp.pad(ew.reshape(_E, _OUT * _C), ((0, 0), (0, 1))).reshape(-1)
    eb_p = jnp.pad(eb, ((0, 0), (0, 1))).reshape(-1)
    return jnp.concatenate([gw_s, gb_s, ew_p, eb_p])


@jax.jit
def _run(x, gw, gb, ew, eb):
    outs = []
    if _H_TC > 0:
        outs.append(_run_tc(x, gw, gb, ew, eb, _H_TC))
    if _H_SC > 0:
        wt = _pack_weights(gw, gb, ew, eb)
        sc = _make_sc(_H_SC, _H_TC)
        B, C, H, W = x.shape
        fin_sc, log_sc = sc(x.reshape(B, C, H * W), wt)
        fin_sc = fin_sc.reshape(B, _OUT, _H_SC, W)
        log_sc = log_sc.reshape(B, C, _H_SC, W)
        outs.append((fin_sc, log_sc))
    if len(outs) == 1:
        return outs[0]
    fin = jnp.concatenate([outs[0][0], outs[1][0]], axis=2)
    log = jnp.concatenate([outs[0][1], outs[1][1]], axis=2)
    return fin, log


def kernel(x, gate_w, gate_b, expert_w, expert_b):
    gw = gate_w.reshape(_C, _C)
    gb = gate_b.reshape(1, _C)
    ew = expert_w.reshape(_E * _OUT, _C)
    eb = expert_b.reshape(_E, _OUT)
    final, logits = _run(x, gw, gb, ew, eb)
    return (final, logits)
